# Initial kernel scaffold; baseline (speedup 1.0000x reference)
import jax
import jax.numpy as jnp
from jax.experimental import pallas as pl


def kernel(x, edge_index, edge_attr, W1, b1, W2, b2):
    raise NotImplementedError("write your pallas kernel here")


# speedup vs baseline: 4.9750x; 4.9750x over previous
import functools

import jax
import jax.numpy as jnp
from jax import lax
from jax.experimental import pallas as pl
from jax.experimental.pallas import tpu as pltpu
from jax.experimental.pallas import tpu_sc as plsc

N = 10000
D = 128
NC = 2
NS = 16
DH = D // NC
LANE = 16
CH = 128
E_PAD = 327680
EPS = E_PAD // NS
CPW = EPS // CH
NPAD = 10112
RPT = NPAD // NS
ZR = 158
BN = 1000
GRID = N // BN

_mesh = plsc.VectorSubcoreMesh(core_axis_name="c", subcore_axis_name="s",
                               num_cores=NC, num_subcores=NS)


def _make_deg(interpret=False):
  return functools.partial(
    pl.kernel,
    out_type=jax.ShapeDtypeStruct((NC, NPAD, LANE), jnp.float32),
    mesh=_mesh,
    scratch_types=[
        pltpu.VMEM((CH,), jnp.int32),
        pltpu.VMEM((CH, LANE), jnp.float32),
        pltpu.VMEM((ZR, LANE), jnp.float32),
        pltpu.VMEM_SHARED((NPAD, LANE), jnp.float32),
    ],
    compiler_params=pltpu.CompilerParams(use_tc_tiling_on_sc=False),
    interpret=interpret,
  )(_deg_body)


def _deg_body(dst_hbm, wbc_hbm, out_hbm, dstv, buf, zbuf, acc):
    c = lax.axis_index("c")
    s = lax.axis_index("s")
    wid = s * NC + c
    zero = jnp.zeros((LANE,), jnp.float32)

    def zrow(i, carry):
        zbuf[i, :] = zero
        return carry

    lax.fori_loop(0, ZR, zrow, 0)
    for r in range(RPT // ZR):
        pltpu.sync_copy(zbuf, acc.at[pl.ds(s * RPT + r * ZR, ZR)])
    plsc.subcore_barrier()

    def chunk(ci, carry):
        base = wid * (E_PAD // (NC * NS)) + ci * CH
        pltpu.sync_copy(dst_hbm.at[pl.ds(base, CH)], dstv)
        pltpu.sync_copy(wbc_hbm.at[pl.ds(base, CH)], buf)
        pltpu.sync_copy(buf, acc.at[dstv], add=True)
        return carry

    lax.fori_loop(0, CPW // NC, chunk, 0)
    plsc.subcore_barrier()
    pltpu.sync_copy(acc.at[pl.ds(s * RPT, RPT)],
                    out_hbm.at[c, pl.ds(s * RPT, RPT)])


def _make_mp(interpret=False):
  return functools.partial(
    pl.kernel,
    out_type=jax.ShapeDtypeStruct((NC, NPAD, DH), jnp.float32),
    mesh=_mesh,
    scratch_types=[
        pltpu.VMEM((CH,), jnp.int32),
        pltpu.VMEM((CH,), jnp.int32),
        pltpu.VMEM((CH, LANE), jnp.float32),
        pltpu.VMEM((CH, DH), jnp.float32),
        pltpu.VMEM((ZR, DH), jnp.float32),
        pltpu.VMEM_SHARED((NPAD, DH), jnp.float32),
        pltpu.SemaphoreType.DMA,
    ],
    compiler_params=pltpu.CompilerParams(use_tc_tiling_on_sc=False),
    interpret=interpret,
  )(_mp_body)


def _mp_body(hp_hbm, src_hbm, dst_hbm, wbc_hbm, out_hbm,
               srcv, dstv, wbuf, rows, zbuf, acc, sem):
    c = lax.axis_index("c")
    s = lax.axis_index("s")
    zero = jnp.zeros((LANE,), jnp.float32)

    def zrow(i, carry):
        for f in range(DH // LANE):
            zbuf[i, pl.ds(f * LANE, LANE)] = zero
        return carry

    lax.fori_loop(0, ZR, zrow, 0)
    for r in range(RPT // ZR):
        pltpu.sync_copy(zbuf, acc.at[pl.ds(s * RPT + r * ZR, ZR)])
    plsc.subcore_barrier()

    def chunk(ci, carry):
        base = s * EPS + ci * CH
        pltpu.sync_copy(src_hbm.at[pl.ds(base, CH)], srcv)
        pltpu.sync_copy(dst_hbm.at[pl.ds(base, CH)], dstv)
        pltpu.sync_copy(wbc_hbm.at[pl.ds(base, CH)], wbuf)
        pltpu.async_copy(hp_hbm.at[c].at[srcv], rows, sem).wait()

        def scale_row(e, carry2):
            wrow = wbuf[e, :]
            for f in range(DH // LANE):
                rows[e, pl.ds(f * LANE, LANE)] = (
                    rows[e, pl.ds(f * LANE, LANE)] * wrow)
            return carry2

        lax.fori_loop(0, CH, scale_row, 0)
        pltpu.sync_copy(rows, acc.at[dstv], add=True)
        return carry

    lax.fori_loop(0, CPW, chunk, 0)
    plsc.subcore_barrier()
    pltpu.sync_copy(acc.at[pl.ds(s * RPT, RPT)],
                    out_hbm.at[c, pl.ds(s * RPT, RPT)])


_deg_kernel = _make_deg()
_mp_kernel = _make_mp()


def _tc1_body(degp_ref, x_ref, w1_ref, dis_ref, hp_ref):
    deg = degp_ref[0, :, 0:1] + degp_ref[1, :, 0:1] + 1.0
    dis = jnp.where(deg > 0, lax.rsqrt(jnp.maximum(deg, 1e-12)), 0.0)
    h = jnp.dot(x_ref[...], w1_ref[...], preferred_element_type=jnp.float32)
    hp = h * dis
    dis_ref[...] = dis
    hp_ref[0] = hp[:, :DH]
    hp_ref[1] = hp[:, DH:]


_tc1 = pl.pallas_call(
    _tc1_body,
    grid=(GRID,),
    in_specs=[
        pl.BlockSpec((NC, BN, LANE), lambda i: (0, i, 0)),
        pl.BlockSpec((BN, D), lambda i: (i, 0)),
        pl.BlockSpec((D, D), lambda i: (0, 0)),
    ],
    out_specs=[
        pl.BlockSpec((BN, 1), lambda i: (i, 0)),
        pl.BlockSpec((NC, BN, DH), lambda i: (0, i, 0)),
    ],
    out_shape=[
        jax.ShapeDtypeStruct((N, 1), jnp.float32),
        jax.ShapeDtypeStruct((NC, N, DH), jnp.float32),
    ],
)


def _tc2_body(p_ref, hp_ref, dis_ref, b_ref, w2_ref, hp2_ref):
    ssum = jnp.concatenate(
        [p_ref[0] + hp_ref[0], p_ref[1] + hp_ref[1]], axis=1)
    o = jnp.maximum(dis_ref[...] * ssum + b_ref[...], 0.0)
    h2 = jnp.dot(o, w2_ref[...], preferred_element_type=jnp.float32)
    hp2 = h2 * dis_ref[...]
    hp2_ref[0] = hp2[:, :DH]
    hp2_ref[1] = hp2[:, DH:]


_tc2 = pl.pallas_call(
    _tc2_body,
    grid=(GRID,),
    in_specs=[
        pl.BlockSpec((NC, BN, DH), lambda i: (0, i, 0)),
        pl.BlockSpec((NC, BN, DH), lambda i: (0, i, 0)),
        pl.BlockSpec((BN, 1), lambda i: (i, 0)),
        pl.BlockSpec((1, D), lambda i: (0, 0)),
        pl.BlockSpec((D, D), lambda i: (0, 0)),
    ],
    out_specs=pl.BlockSpec((NC, BN, DH), lambda i: (0, i, 0)),
    out_shape=jax.ShapeDtypeStruct((NC, N, DH), jnp.float32),
)


def _tc3_body(p_ref, hp_ref, dis_ref, b_ref, out_ref):
    ssum = jnp.concatenate(
        [p_ref[0] + hp_ref[0], p_ref[1] + hp_ref[1]], axis=1)
    out_ref[...] = dis_ref[...] * ssum + b_ref[...]


_tc3 = pl.pallas_call(
    _tc3_body,
    grid=(GRID,),
    in_specs=[
        pl.BlockSpec((NC, BN, DH), lambda i: (0, i, 0)),
        pl.BlockSpec((NC, BN, DH), lambda i: (0, i, 0)),
        pl.BlockSpec((BN, 1), lambda i: (i, 0)),
        pl.BlockSpec((1, D), lambda i: (0, 0)),
    ],
    out_specs=pl.BlockSpec((BN, D), lambda i: (i, 0)),
    out_shape=jax.ShapeDtypeStruct((N, D), jnp.float32),
)


def kernel(x, edge_index, edge_attr, W1, b1, W2, b2):
    src = edge_index[0]
    dst = edge_index[1]
    pad = E_PAD - src.shape[0]
    src_p = jnp.concatenate([src, jnp.zeros((pad,), src.dtype)])
    dst_p = jnp.concatenate([dst, jnp.zeros((pad,), dst.dtype)])
    w_p = jnp.concatenate([edge_attr, jnp.zeros((pad,), edge_attr.dtype)])
    w_bc = jnp.broadcast_to(w_p[:, None], (E_PAD, LANE))
    degp = _deg_kernel(dst_p, w_bc)
    dis, hp1 = _tc1(degp, x, W1)
    p1 = _mp_kernel(hp1, src_p, dst_p, w_bc)
    hp2 = _tc2(p1, hp1, dis, b1.reshape(1, D), W2)
    p2 = _mp_kernel(hp2, src_p, dst_p, w_bc)
    return _tc3(p2, hp2, dis, b2.reshape(1, D))


# speedup vs baseline: 6.8248x; 1.3718x over previous
import functools

import jax
import jax.numpy as jnp
from jax import lax
from jax.experimental import pallas as pl
from jax.experimental.pallas import tpu as pltpu
from jax.experimental.pallas import tpu_sc as plsc

N = 10000
D = 128
NC = 2
NS = 16
DH = D // NC
LANE = 16
CH = 128
E_PAD = 327680
EPS = E_PAD // NS
CPW = EPS // CH
NPAD = 10112
RPT = NPAD // NS
ZR = 158
BN = 1000
GRID = N // BN

_mesh = plsc.VectorSubcoreMesh(core_axis_name="c", subcore_axis_name="s",
                               num_cores=NC, num_subcores=NS)


def _make_deg(interpret=False):
  return functools.partial(
    pl.kernel,
    out_type=jax.ShapeDtypeStruct((NC, NPAD, LANE), jnp.float32),
    mesh=_mesh,
    scratch_types=[
        pltpu.VMEM((CH,), jnp.int32),
        pltpu.VMEM((CH, LANE), jnp.float32),
        pltpu.VMEM((ZR, LANE), jnp.float32),
        pltpu.VMEM_SHARED((NPAD, LANE), jnp.float32),
    ],
    compiler_params=pltpu.CompilerParams(use_tc_tiling_on_sc=False),
    interpret=interpret,
  )(_deg_body)


def _deg_body(dst_hbm, wbc_hbm, out_hbm, dstv, buf, zbuf, acc):
    c = lax.axis_index("c")
    s = lax.axis_index("s")
    wid = s * NC + c
    zero = jnp.zeros((LANE,), jnp.float32)

    def zrow(i, carry):
        zbuf[i, :] = zero
        return carry

    lax.fori_loop(0, ZR, zrow, 0)
    for r in range(RPT // ZR):
        pltpu.sync_copy(zbuf, acc.at[pl.ds(s * RPT + r * ZR, ZR)])
    plsc.subcore_barrier()

    def chunk(ci, carry):
        base = wid * (E_PAD // (NC * NS)) + ci * CH
        pltpu.sync_copy(dst_hbm.at[pl.ds(base, CH)], dstv)
        pltpu.sync_copy(wbc_hbm.at[pl.ds(base, CH)], buf)
        pltpu.sync_copy(buf, acc.at[dstv], add=True)
        return carry

    lax.fori_loop(0, CPW // NC, chunk, 0)
    plsc.subcore_barrier()
    pltpu.sync_copy(acc.at[pl.ds(s * RPT, RPT)],
                    out_hbm.at[c, pl.ds(s * RPT, RPT)])


def _make_mp(interpret=False):
  return functools.partial(
    pl.kernel,
    out_type=jax.ShapeDtypeStruct((NC, NPAD, DH), jnp.float32),
    mesh=_mesh,
    scratch_types=[
        pltpu.VMEM((CPW, CH), jnp.int32),
        pltpu.VMEM((CPW, CH), jnp.int32),
        pltpu.VMEM((CPW, CH), jnp.float32),
        pltpu.VMEM((CH, DH), jnp.float32),
        pltpu.VMEM((CH, DH), jnp.float32),
        pltpu.VMEM((ZR, DH), jnp.float32),
        pltpu.VMEM_SHARED((NPAD, DH), jnp.float32),
        pltpu.SemaphoreType.DMA,
        pltpu.SemaphoreType.DMA,
        pltpu.SemaphoreType.DMA,
        pltpu.SemaphoreType.DMA,
    ],
    compiler_params=pltpu.CompilerParams(use_tc_tiling_on_sc=False),
    interpret=interpret,
  )(_mp_body)


def _mp_body(hp_hbm, src_hbm, dst_hbm, w_hbm, out_hbm,
             srca, dsta, wa, rows0, rows1, zbuf, acc,
             gsem0, gsem1, ssem0, ssem1):
    c = lax.axis_index("c")
    s = lax.axis_index("s")
    zero = jnp.zeros((LANE,), jnp.float32)

    def zrow(i, carry):
        for f in range(DH // LANE):
            zbuf[i, pl.ds(f * LANE, LANE)] = zero
        return carry

    lax.fori_loop(0, ZR, zrow, 0)
    pltpu.sync_copy(src_hbm.at[s], srca)
    pltpu.sync_copy(dst_hbm.at[s], dsta)
    pltpu.sync_copy(w_hbm.at[s], wa)
    for r in range(RPT // ZR):
        pltpu.sync_copy(zbuf, acc.at[pl.ds(s * RPT + r * ZR, ZR)])
    plsc.subcore_barrier()

    hp = hp_hbm.at[c]
    bufs = ((rows0, gsem0, ssem0), (rows1, gsem1, ssem1))
    take_idx = [jnp.full((LANE,), jj, jnp.int32) for jj in range(LANE)]

    pltpu.async_copy(hp.at[srca.at[0]], rows0, gsem0)

    def scale(rows, j):
        def grp(g, carry2):
            w16 = wa[j, pl.ds(g * LANE, LANE)]
            for jj in range(LANE):
                wb = w16[take_idx[jj]]
                e = g * LANE + jj
                for f in range(DH // LANE):
                    rows[e, pl.ds(f * LANE, LANE)] = (
                        rows[e, pl.ds(f * LANE, LANE)] * wb)
            return carry2

        lax.fori_loop(0, CH // LANE, grp, 0)

    def super2(j2, carry):
        for b, (rows, gsem, ssem) in enumerate(bufs):
            j = 2 * j2 + b
            nrows, ngsem, _ = bufs[1 - b]
            pltpu.make_async_copy(hp.at[srca.at[j]], rows, gsem).wait()

            @pl.when(j + 1 < CPW)
            def _():
                pltpu.async_copy(hp.at[srca.at[j + 1]], nrows, ngsem)

            scale(rows, j)
            pltpu.sync_copy(rows, acc.at[dsta.at[j]], add=True)
        return carry

    lax.fori_loop(0, CPW // 2, super2, 0)
    plsc.subcore_barrier()
    pltpu.sync_copy(acc.at[pl.ds(s * RPT, RPT)],
                    out_hbm.at[c, pl.ds(s * RPT, RPT)])


_deg_kernel = _make_deg()
_mp_kernel = _make_mp()


def _tc1_body(degp_ref, x_ref, w1_ref, dis_ref, hp_ref):
    deg = degp_ref[0, :, 0:1] + degp_ref[1, :, 0:1] + 1.0
    dis = jnp.where(deg > 0, lax.rsqrt(jnp.maximum(deg, 1e-12)), 0.0)
    h = jnp.dot(x_ref[...], w1_ref[...], preferred_element_type=jnp.float32)
    hp = h * dis
    dis_ref[...] = dis
    hp_ref[0] = hp[:, :DH]
    hp_ref[1] = hp[:, DH:]


_tc1 = pl.pallas_call(
    _tc1_body,
    grid=(GRID,),
    in_specs=[
        pl.BlockSpec((NC, BN, LANE), lambda i: (0, i, 0)),
        pl.BlockSpec((BN, D), lambda i: (i, 0)),
        pl.BlockSpec((D, D), lambda i: (0, 0)),
    ],
    out_specs=[
        pl.BlockSpec((BN, 1), lambda i: (i, 0)),
        pl.BlockSpec((NC, BN, DH), lambda i: (0, i, 0)),
    ],
    out_shape=[
        jax.ShapeDtypeStruct((N, 1), jnp.float32),
        jax.ShapeDtypeStruct((NC, N, DH), jnp.float32),
    ],
)


def _tc2_body(p_ref, hp_ref, dis_ref, b_ref, w2_ref, hp2_ref):
    ssum = jnp.concatenate(
        [p_ref[0] + hp_ref[0], p_ref[1] + hp_ref[1]], axis=1)
    o = jnp.maximum(dis_ref[...] * ssum + b_ref[...], 0.0)
    h2 = jnp.dot(o, w2_ref[...], preferred_element_type=jnp.float32)
    hp2 = h2 * dis_ref[...]
    hp2_ref[0] = hp2[:, :DH]
    hp2_ref[1] = hp2[:, DH:]


_tc2 = pl.pallas_call(
    _tc2_body,
    grid=(GRID,),
    in_specs=[
        pl.BlockSpec((NC, BN, DH), lambda i: (0, i, 0)),
        pl.BlockSpec((NC, BN, DH), lambda i: (0, i, 0)),
        pl.BlockSpec((BN, 1), lambda i: (i, 0)),
        pl.BlockSpec((1, D), lambda i: (0, 0)),
        pl.BlockSpec((D, D), lambda i: (0, 0)),
    ],
    out_specs=pl.BlockSpec((NC, BN, DH), lambda i: (0, i, 0)),
    out_shape=jax.ShapeDtypeStruct((NC, N, DH), jnp.float32),
)


def _tc3_body(p_ref, hp_ref, dis_ref, b_ref, out_ref):
    ssum = jnp.concatenate(
        [p_ref[0] + hp_ref[0], p_ref[1] + hp_ref[1]], axis=1)
    out_ref[...] = dis_ref[...] * ssum + b_ref[...]


_tc3 = pl.pallas_call(
    _tc3_body,
    grid=(GRID,),
    in_specs=[
        pl.BlockSpec((NC, BN, DH), lambda i: (0, i, 0)),
        pl.BlockSpec((NC, BN, DH), lambda i: (0, i, 0)),
        pl.BlockSpec((BN, 1), lambda i: (i, 0)),
        pl.BlockSpec((1, D), lambda i: (0, 0)),
    ],
    out_specs=pl.BlockSpec((BN, D), lambda i: (i, 0)),
    out_shape=jax.ShapeDtypeStruct((N, D), jnp.float32),
)


def kernel(x, edge_index, edge_attr, W1, b1, W2, b2):
    src = edge_index[0]
    dst = edge_index[1]
    pad = E_PAD - src.shape[0]
    src_p = jnp.concatenate([src, jnp.zeros((pad,), src.dtype)])
    dst_p = jnp.concatenate([dst, jnp.zeros((pad,), dst.dtype)])
    w_p = jnp.concatenate([edge_attr, jnp.zeros((pad,), edge_attr.dtype)])
    w_bc = jnp.broadcast_to(w_p[:, None], (E_PAD, LANE))
    src3 = src_p.reshape(NS, CPW, CH)
    dst3 = dst_p.reshape(NS, CPW, CH)
    w3 = w_p.reshape(NS, CPW, CH)
    degp = _deg_kernel(dst_p, w_bc)
    dis, hp1 = _tc1(degp, x, W1)
    p1 = _mp_kernel(hp1, src3, dst3, w3)
    hp2 = _tc2(p1, hp1, dis, b1.reshape(1, D), W2)
    p2 = _mp_kernel(hp2, src3, dst3, w3)
    return _tc3(p2, hp2, dis, b2.reshape(1, D))


# speedup vs baseline: 8.5816x; 1.2574x over previous
import functools

import jax
import jax.numpy as jnp
from jax import lax
from jax.experimental import pallas as pl
from jax.experimental.pallas import tpu as pltpu
from jax.experimental.pallas import tpu_sc as plsc

N = 10000
D = 128
NC = 2
NS = 16
DH = D // NC
LANE = 16
CH = 128
E_PAD = 327680
EPS = E_PAD // NS
CPW = EPS // CH
NPAD = 10112
RPT = NPAD // NS
ZR = 158
BN = 1000
GRID = N // BN

_mesh = plsc.VectorSubcoreMesh(core_axis_name="c", subcore_axis_name="s",
                               num_cores=NC, num_subcores=NS)


def _make_deg(interpret=False):
  return functools.partial(
    pl.kernel,
    out_type=jax.ShapeDtypeStruct((NC, NPAD, LANE), jnp.float32),
    mesh=_mesh,
    scratch_types=[
        pltpu.VMEM((CH,), jnp.int32),
        pltpu.VMEM((CH, LANE), jnp.float32),
        pltpu.VMEM((ZR, LANE), jnp.float32),
        pltpu.VMEM_SHARED((NPAD, LANE), jnp.float32),
    ],
    compiler_params=pltpu.CompilerParams(use_tc_tiling_on_sc=False),
    interpret=interpret,
  )(_deg_body)


def _deg_body(dst_hbm, wbc_hbm, out_hbm, dstv, buf, zbuf, acc):
    c = lax.axis_index("c")
    s = lax.axis_index("s")
    wid = s * NC + c
    zero = jnp.zeros((LANE,), jnp.float32)

    def zrow(i, carry):
        zbuf[i, :] = zero
        return carry

    lax.fori_loop(0, ZR, zrow, 0)
    for r in range(RPT // ZR):
        pltpu.sync_copy(zbuf, acc.at[pl.ds(s * RPT + r * ZR, ZR)])
    plsc.subcore_barrier()

    def chunk(ci, carry):
        base = wid * (E_PAD // (NC * NS)) + ci * CH
        pltpu.sync_copy(dst_hbm.at[pl.ds(base, CH)], dstv)
        pltpu.sync_copy(wbc_hbm.at[pl.ds(base, CH)], buf)
        pltpu.sync_copy(buf, acc.at[dstv], add=True)
        return carry

    lax.fori_loop(0, CPW // NC, chunk, 0)
    plsc.subcore_barrier()
    pltpu.sync_copy(acc.at[pl.ds(s * RPT, RPT)],
                    out_hbm.at[c, pl.ds(s * RPT, RPT)])


def _make_mp(interpret=False):
  return functools.partial(
    pl.kernel,
    out_type=jax.ShapeDtypeStruct((NC, NPAD, DH), jnp.float32),
    mesh=_mesh,
    scratch_types=[
        pltpu.VMEM((CPW, CH), jnp.int32),
        pltpu.VMEM((CPW, CH), jnp.int32),
        pltpu.VMEM((CPW, CH), jnp.float32),
        pltpu.VMEM((CH, DH), jnp.float32),
        pltpu.VMEM((CH, DH), jnp.float32),
        pltpu.VMEM((ZR, DH), jnp.float32),
        pltpu.VMEM_SHARED((NPAD, DH), jnp.float32),
        pltpu.SemaphoreType.DMA,
        pltpu.SemaphoreType.DMA,
        pltpu.SemaphoreType.DMA,
        pltpu.SemaphoreType.DMA,
    ],
    compiler_params=pltpu.CompilerParams(use_tc_tiling_on_sc=False),
    interpret=interpret,
  )(_mp_body)


def _mp_body(hp_hbm, src_hbm, dst_hbm, w_hbm, out_hbm,
             srca, dsta, wa, rows0, rows1, zbuf, acc,
             gsem0, gsem1, ssem0, ssem1):
    c = lax.axis_index("c")
    s = lax.axis_index("s")
    zero = jnp.zeros((LANE,), jnp.float32)

    def zrow(i, carry):
        for f in range(DH // LANE):
            zbuf[i, pl.ds(f * LANE, LANE)] = zero
        return carry

    lax.fori_loop(0, ZR, zrow, 0)
    pltpu.sync_copy(src_hbm.at[s], srca)
    pltpu.sync_copy(dst_hbm.at[s], dsta)
    pltpu.sync_copy(w_hbm.at[s], wa)
    for r in range(RPT // ZR):
        pltpu.sync_copy(zbuf, acc.at[pl.ds(s * RPT + r * ZR, ZR)])
    plsc.subcore_barrier()

    hp = hp_hbm.at[c]
    bufs = ((rows0, gsem0, ssem0), (rows1, gsem1, ssem1))
    take_idx = [jnp.full((LANE,), jj, jnp.int32) for jj in range(LANE)]

    pltpu.async_copy(hp.at[srca.at[0]], rows0, gsem0)

    def scale(rows, j):
        @plsc.parallel_loop(0, CH // LANE, unroll=2)
        def grp(g):
            w16 = wa[j, pl.ds(g * LANE, LANE)]
            for jj in range(LANE):
                wb = w16[take_idx[jj]]
                e = g * LANE + jj
                for f in range(DH // LANE):
                    rows[e, pl.ds(f * LANE, LANE)] = (
                        rows[e, pl.ds(f * LANE, LANE)] * wb)

    def super2(j2, carry):
        for b, (rows, gsem, ssem) in enumerate(bufs):
            j = 2 * j2 + b
            nrows, ngsem, _ = bufs[1 - b]
            pltpu.make_async_copy(hp.at[srca.at[j]], rows, gsem).wait()

            @pl.when(j + 1 < CPW)
            def _():
                pltpu.async_copy(hp.at[srca.at[j + 1]], nrows, ngsem)

            scale(rows, j)
            pltpu.sync_copy(rows, acc.at[dsta.at[j]], add=True)
        return carry

    lax.fori_loop(0, CPW // 2, super2, 0)
    plsc.subcore_barrier()
    pltpu.sync_copy(acc.at[pl.ds(s * RPT, RPT)],
                    out_hbm.at[c, pl.ds(s * RPT, RPT)])


_deg_kernel = _make_deg()
_mp_kernel = _make_mp()


def _tc1_body(degp_ref, x_ref, w1_ref, dis_ref, hp_ref):
    deg = degp_ref[0, :, 0:1] + degp_ref[1, :, 0:1] + 1.0
    dis = jnp.where(deg > 0, lax.rsqrt(jnp.maximum(deg, 1e-12)), 0.0)
    h = jnp.dot(x_ref[...], w1_ref[...], preferred_element_type=jnp.float32)
    hp = h * dis
    dis_ref[...] = dis
    hp_ref[0] = hp[:, :DH]
    hp_ref[1] = hp[:, DH:]


_tc1 = pl.pallas_call(
    _tc1_body,
    grid=(GRID,),
    in_specs=[
        pl.BlockSpec((NC, BN, LANE), lambda i: (0, i, 0)),
        pl.BlockSpec((BN, D), lambda i: (i, 0)),
        pl.BlockSpec((D, D), lambda i: (0, 0)),
    ],
    out_specs=[
        pl.BlockSpec((BN, 1), lambda i: (i, 0)),
        pl.BlockSpec((NC, BN, DH), lambda i: (0, i, 0)),
    ],
    out_shape=[
        jax.ShapeDtypeStruct((N, 1), jnp.float32),
        jax.ShapeDtypeStruct((NC, N, DH), jnp.float32),
    ],
)


def _tc2_body(p_ref, hp_ref, dis_ref, b_ref, w2_ref, hp2_ref):
    ssum = jnp.concatenate(
        [p_ref[0] + hp_ref[0], p_ref[1] + hp_ref[1]], axis=1)
    o = jnp.maximum(dis_ref[...] * ssum + b_ref[...], 0.0)
    h2 = jnp.dot(o, w2_ref[...], preferred_element_type=jnp.float32)
    hp2 = h2 * dis_ref[...]
    hp2_ref[0] = hp2[:, :DH]
    hp2_ref[1] = hp2[:, DH:]


_tc2 = pl.pallas_call(
    _tc2_body,
    grid=(GRID,),
    in_specs=[
        pl.BlockSpec((NC, BN, DH), lambda i: (0, i, 0)),
        pl.BlockSpec((NC, BN, DH), lambda i: (0, i, 0)),
        pl.BlockSpec((BN, 1), lambda i: (i, 0)),
        pl.BlockSpec((1, D), lambda i: (0, 0)),
        pl.BlockSpec((D, D), lambda i: (0, 0)),
    ],
    out_specs=pl.BlockSpec((NC, BN, DH), lambda i: (0, i, 0)),
    out_shape=jax.ShapeDtypeStruct((NC, N, DH), jnp.float32),
)


def _tc3_body(p_ref, hp_ref, dis_ref, b_ref, out_ref):
    ssum = jnp.concatenate(
        [p_ref[0] + hp_ref[0], p_ref[1] + hp_ref[1]], axis=1)
    out_ref[...] = dis_ref[...] * ssum + b_ref[...]


_tc3 = pl.pallas_call(
    _tc3_body,
    grid=(GRID,),
    in_specs=[
        pl.BlockSpec((NC, BN, DH), lambda i: (0, i, 0)),
        pl.BlockSpec((NC, BN, DH), lambda i: (0, i, 0)),
        pl.BlockSpec((BN, 1), lambda i: (i, 0)),
        pl.BlockSpec((1, D), lambda i: (0, 0)),
    ],
    out_specs=pl.BlockSpec((BN, D), lambda i: (i, 0)),
    out_shape=jax.ShapeDtypeStruct((N, D), jnp.float32),
)


def kernel(x, edge_index, edge_attr, W1, b1, W2, b2):
    src = edge_index[0]
    dst = edge_index[1]
    pad = E_PAD - src.shape[0]
    src_p = jnp.concatenate([src, jnp.zeros((pad,), src.dtype)])
    dst_p = jnp.concatenate([dst, jnp.zeros((pad,), dst.dtype)])
    w_p = jnp.concatenate([edge_attr, jnp.zeros((pad,), edge_attr.dtype)])
    w_bc = jnp.broadcast_to(w_p[:, None], (E_PAD, LANE))
    src3 = src_p.reshape(NS, CPW, CH)
    dst3 = dst_p.reshape(NS, CPW, CH)
    w3 = w_p.reshape(NS, CPW, CH)
    degp = _deg_kernel(dst_p, w_bc)
    dis, hp1 = _tc1(degp, x, W1)
    p1 = _mp_kernel(hp1, src3, dst3, w3)
    hp2 = _tc2(p1, hp1, dis, b1.reshape(1, D), W2)
    p2 = _mp_kernel(hp2, src3, dst3, w3)
    return _tc3(p2, hp2, dis, b2.reshape(1, D))


# speedup vs baseline: 9.0451x; 1.0540x over previous
import functools

import jax
import jax.numpy as jnp
from jax import lax
from jax.experimental import pallas as pl
from jax.experimental.pallas import tpu as pltpu
from jax.experimental.pallas import tpu_sc as plsc

N = 10000
D = 128
NC = 2
NS = 16
DH = D // NC
LANE = 16
CH = 128
E_PAD = 327680
EPS = E_PAD // NS
CPW = EPS // CH
NPAD = 10112
RPT = NPAD // NS
ZR = 158
BN = 1000
GRID = N // BN

_mesh = plsc.VectorSubcoreMesh(core_axis_name="c", subcore_axis_name="s",
                               num_cores=NC, num_subcores=NS)


def _make_deg(interpret=False):
  return functools.partial(
    pl.kernel,
    out_type=jax.ShapeDtypeStruct((NC, NPAD, LANE), jnp.float32),
    mesh=_mesh,
    scratch_types=[
        pltpu.VMEM((CPW // NC, CH), jnp.int32),
        pltpu.VMEM((CH, LANE), jnp.float32),
        pltpu.VMEM((CH, LANE), jnp.float32),
        pltpu.VMEM((ZR, LANE), jnp.float32),
        pltpu.VMEM_SHARED((NPAD, LANE), jnp.float32),
        pltpu.SemaphoreType.DMA,
        pltpu.SemaphoreType.DMA,
    ],
    compiler_params=pltpu.CompilerParams(use_tc_tiling_on_sc=False),
    interpret=interpret,
  )(_deg_body)


def _deg_body(dst_hbm, wbc_hbm, out_hbm, dsta, buf0, buf1, zbuf, acc, w0, w1):
    c = lax.axis_index("c")
    s = lax.axis_index("s")
    cpw = CPW // NC
    zero = jnp.zeros((LANE,), jnp.float32)

    def zrow(i, carry):
        zbuf[i, :] = zero
        return carry

    lax.fori_loop(0, ZR, zrow, 0)
    pltpu.sync_copy(dst_hbm.at[s, pl.ds(c * cpw, cpw)], dsta)
    for r in range(RPT // ZR):
        pltpu.sync_copy(zbuf, acc.at[pl.ds(s * RPT + r * ZR, ZR)])
    plsc.subcore_barrier()

    base_e = s * EPS + c * (EPS // NC)
    bufs = ((buf0, w0), (buf1, w1))
    pltpu.async_copy(wbc_hbm.at[pl.ds(base_e, CH)], buf0, w0)

    def chunk2(j2, carry):
        for b, (buf, wsem) in enumerate(bufs):
            j = 2 * j2 + b
            nbuf, nwsem = bufs[1 - b]
            pltpu.make_async_copy(
                wbc_hbm.at[pl.ds(base_e + j * CH, CH)], buf, wsem).wait()

            @pl.when(j + 1 < cpw)
            def _():
                pltpu.async_copy(
                    wbc_hbm.at[pl.ds(base_e + (j + 1) * CH, CH)], nbuf, nwsem)

            pltpu.sync_copy(buf, acc.at[dsta.at[j]], add=True)
        return carry

    lax.fori_loop(0, cpw // 2, chunk2, 0)
    plsc.subcore_barrier()
    pltpu.sync_copy(acc.at[pl.ds(s * RPT, RPT)],
                    out_hbm.at[c, pl.ds(s * RPT, RPT)])


def _make_mp(interpret=False):
  return functools.partial(
    pl.kernel,
    out_type=jax.ShapeDtypeStruct((NC, NPAD, DH), jnp.float32),
    mesh=_mesh,
    scratch_types=[
        pltpu.VMEM((CPW, CH), jnp.int32),
        pltpu.VMEM((CPW, CH), jnp.int32),
        pltpu.VMEM((CPW, CH), jnp.float32),
        pltpu.VMEM((CH, DH), jnp.float32),
        pltpu.VMEM((CH, DH), jnp.float32),
        pltpu.VMEM((ZR, DH), jnp.float32),
        pltpu.VMEM_SHARED((NPAD, DH), jnp.float32),
        pltpu.SemaphoreType.DMA,
        pltpu.SemaphoreType.DMA,
        pltpu.SemaphoreType.DMA,
        pltpu.SemaphoreType.DMA,
    ],
    compiler_params=pltpu.CompilerParams(use_tc_tiling_on_sc=False),
    interpret=interpret,
  )(_mp_body)


def _mp_body(hp_hbm, src_hbm, dst_hbm, w_hbm, out_hbm,
             srca, dsta, wa, rows0, rows1, zbuf, acc,
             gsem0, gsem1, ssem0, ssem1):
    c = lax.axis_index("c")
    s = lax.axis_index("s")
    zero = jnp.zeros((LANE,), jnp.float32)

    def zrow(i, carry):
        for f in range(DH // LANE):
            zbuf[i, pl.ds(f * LANE, LANE)] = zero
        return carry

    lax.fori_loop(0, ZR, zrow, 0)
    pltpu.sync_copy(src_hbm.at[s], srca)
    pltpu.sync_copy(dst_hbm.at[s], dsta)
    pltpu.sync_copy(w_hbm.at[s], wa)
    for r in range(RPT // ZR):
        pltpu.sync_copy(zbuf, acc.at[pl.ds(s * RPT + r * ZR, ZR)])
    plsc.subcore_barrier()

    hp = hp_hbm.at[c]
    bufs = ((rows0, gsem0, ssem0), (rows1, gsem1, ssem1))
    take_idx = [jnp.full((LANE,), jj, jnp.int32) for jj in range(LANE)]

    pltpu.async_copy(hp.at[srca.at[0]], rows0, gsem0)

    def scale(rows, j):
        @plsc.parallel_loop(0, CH // LANE, unroll=4)
        def grp(g):
            w16 = wa[j, pl.ds(g * LANE, LANE)]
            for jj in range(LANE):
                wb = w16[take_idx[jj]]
                e = g * LANE + jj
                for f in range(DH // LANE):
                    rows[e, pl.ds(f * LANE, LANE)] = (
                        rows[e, pl.ds(f * LANE, LANE)] * wb)

    def super2(j2, carry):
        for b, (rows, gsem, ssem) in enumerate(bufs):
            j = 2 * j2 + b
            nrows, ngsem, _ = bufs[1 - b]
            pltpu.make_async_copy(hp.at[srca.at[j]], rows, gsem).wait()

            @pl.when(j + 1 < CPW)
            def _():
                pltpu.async_copy(hp.at[srca.at[j + 1]], nrows, ngsem)

            scale(rows, j)
            pltpu.sync_copy(rows, acc.at[dsta.at[j]], add=True)
        return carry

    lax.fori_loop(0, CPW // 2, super2, 0)
    plsc.subcore_barrier()
    pltpu.sync_copy(acc.at[pl.ds(s * RPT, RPT)],
                    out_hbm.at[c, pl.ds(s * RPT, RPT)])


_deg_kernel = _make_deg()
_mp_kernel = _make_mp()


def _tc1_body(degp_ref, x_ref, w1_ref, dis_ref, hp_ref):
    deg = degp_ref[0, :, 0:1] + degp_ref[1, :, 0:1] + 1.0
    dis = jnp.where(deg > 0, lax.rsqrt(jnp.maximum(deg, 1e-12)), 0.0)
    h = jnp.dot(x_ref[...], w1_ref[...], preferred_element_type=jnp.float32)
    hp = h * dis
    dis_ref[...] = dis
    hp_ref[0] = hp[:, :DH]
    hp_ref[1] = hp[:, DH:]


_tc1 = pl.pallas_call(
    _tc1_body,
    grid=(GRID,),
    in_specs=[
        pl.BlockSpec((NC, BN, LANE), lambda i: (0, i, 0)),
        pl.BlockSpec((BN, D), lambda i: (i, 0)),
        pl.BlockSpec((D, D), lambda i: (0, 0)),
    ],
    out_specs=[
        pl.BlockSpec((BN, 1), lambda i: (i, 0)),
        pl.BlockSpec((NC, BN, DH), lambda i: (0, i, 0)),
    ],
    out_shape=[
        jax.ShapeDtypeStruct((N, 1), jnp.float32),
        jax.ShapeDtypeStruct((NC, N, DH), jnp.float32),
    ],
)


def _tc2_body(p_ref, hp_ref, dis_ref, b_ref, w2_ref, hp2_ref):
    ssum = jnp.concatenate(
        [p_ref[0] + hp_ref[0], p_ref[1] + hp_ref[1]], axis=1)
    o = jnp.maximum(dis_ref[...] * ssum + b_ref[...], 0.0)
    h2 = jnp.dot(o, w2_ref[...], preferred_element_type=jnp.float32)
    hp2 = h2 * dis_ref[...]
    hp2_ref[0] = hp2[:, :DH]
    hp2_ref[1] = hp2[:, DH:]


_tc2 = pl.pallas_call(
    _tc2_body,
    grid=(GRID,),
    in_specs=[
        pl.BlockSpec((NC, BN, DH), lambda i: (0, i, 0)),
        pl.BlockSpec((NC, BN, DH), lambda i: (0, i, 0)),
        pl.BlockSpec((BN, 1), lambda i: (i, 0)),
        pl.BlockSpec((1, D), lambda i: (0, 0)),
        pl.BlockSpec((D, D), lambda i: (0, 0)),
    ],
    out_specs=pl.BlockSpec((NC, BN, DH), lambda i: (0, i, 0)),
    out_shape=jax.ShapeDtypeStruct((NC, N, DH), jnp.float32),
)


def _tc3_body(p_ref, hp_ref, dis_ref, b_ref, out_ref):
    ssum = jnp.concatenate(
        [p_ref[0] + hp_ref[0], p_ref[1] + hp_ref[1]], axis=1)
    out_ref[...] = dis_ref[...] * ssum + b_ref[...]


_tc3 = pl.pallas_call(
    _tc3_body,
    grid=(GRID,),
    in_specs=[
        pl.BlockSpec((NC, BN, DH), lambda i: (0, i, 0)),
        pl.BlockSpec((NC, BN, DH), lambda i: (0, i, 0)),
        pl.BlockSpec((BN, 1), lambda i: (i, 0)),
        pl.BlockSpec((1, D), lambda i: (0, 0)),
    ],
    out_specs=pl.BlockSpec((BN, D), lambda i: (i, 0)),
    out_shape=jax.ShapeDtypeStruct((N, D), jnp.float32),
)


def kernel(x, edge_index, edge_attr, W1, b1, W2, b2):
    src = edge_index[0]
    dst = edge_index[1]
    pad = E_PAD - src.shape[0]
    src_p = jnp.concatenate([src, jnp.zeros((pad,), src.dtype)])
    dst_p = jnp.concatenate([dst, jnp.zeros((pad,), dst.dtype)])
    w_p = jnp.concatenate([edge_attr, jnp.zeros((pad,), edge_attr.dtype)])
    w_bc = jnp.broadcast_to(w_p[:, None], (E_PAD, LANE))
    src3 = src_p.reshape(NS, CPW, CH)
    dst3 = dst_p.reshape(NS, CPW, CH)
    w3 = w_p.reshape(NS, CPW, CH)
    degp = _deg_kernel(dst3, w_bc)
    dis, hp1 = _tc1(degp, x, W1)
    p1 = _mp_kernel(hp1, src3, dst3, w3)
    hp2 = _tc2(p1, hp1, dis, b1.reshape(1, D), W2)
    p2 = _mp_kernel(hp2, src3, dst3, w3)
    return _tc3(p2, hp2, dis, b2.reshape(1, D))


# speedup vs baseline: 9.8047x; 1.0840x over previous
import functools

import jax
import jax.numpy as jnp
from jax import lax
from jax.experimental import pallas as pl
from jax.experimental.pallas import tpu as pltpu
from jax.experimental.pallas import tpu_sc as plsc

N = 10000
D = 128
NC = 2
NS = 16
DH = D // NC
LANE = 16
CH = 128
E_PAD = 327680
EPS = E_PAD // NS
CPW = EPS // CH
NPAD = 10112
RPT = NPAD // NS
ZR = 158
BN = 1000
GRID = N // BN

_mesh = plsc.VectorSubcoreMesh(core_axis_name="c", subcore_axis_name="s",
                               num_cores=NC, num_subcores=NS)


def _make_deg(interpret=False):
  return functools.partial(
    pl.kernel,
    out_type=jax.ShapeDtypeStruct((NC, NPAD, LANE), jnp.float32),
    mesh=_mesh,
    scratch_types=[
        pltpu.VMEM((CPW // NC, CH), jnp.int32),
        pltpu.VMEM((CH, LANE), jnp.float32),
        pltpu.VMEM((CH, LANE), jnp.float32),
        pltpu.VMEM((ZR, LANE), jnp.float32),
        pltpu.VMEM_SHARED((NPAD, LANE), jnp.float32),
        pltpu.SemaphoreType.DMA,
        pltpu.SemaphoreType.DMA,
    ],
    compiler_params=pltpu.CompilerParams(use_tc_tiling_on_sc=False),
    interpret=interpret,
  )(_deg_body)


def _deg_body(dst_hbm, wbc_hbm, out_hbm, dsta, buf0, buf1, zbuf, acc, w0, w1):
    c = lax.axis_index("c")
    s = lax.axis_index("s")
    cpw = CPW // NC
    zero = jnp.zeros((LANE,), jnp.float32)

    def zrow(i, carry):
        zbuf[i, :] = zero
        return carry

    lax.fori_loop(0, ZR, zrow, 0)
    pltpu.sync_copy(dst_hbm.at[s, pl.ds(c * cpw, cpw)], dsta)
    for r in range(RPT // ZR):
        pltpu.sync_copy(zbuf, acc.at[pl.ds(s * RPT + r * ZR, ZR)])
    plsc.subcore_barrier()

    base_e = s * EPS + c * (EPS // NC)
    bufs = ((buf0, w0), (buf1, w1))
    pltpu.async_copy(wbc_hbm.at[pl.ds(base_e, CH)], buf0, w0)

    def chunk2(j2, carry):
        for b, (buf, wsem) in enumerate(bufs):
            j = 2 * j2 + b
            nbuf, nwsem = bufs[1 - b]
            pltpu.make_async_copy(
                wbc_hbm.at[pl.ds(base_e + j * CH, CH)], buf, wsem).wait()

            @pl.when(j + 1 < cpw)
            def _():
                pltpu.async_copy(
                    wbc_hbm.at[pl.ds(base_e + (j + 1) * CH, CH)], nbuf, nwsem)

            pltpu.sync_copy(buf, acc.at[dsta.at[j]], add=True)
        return carry

    lax.fori_loop(0, cpw // 2, chunk2, 0)
    plsc.subcore_barrier()
    pltpu.sync_copy(acc.at[pl.ds(s * RPT, RPT)],
                    out_hbm.at[c, pl.ds(s * RPT, RPT)])


def _make_mp(interpret=False):
  return functools.partial(
    pl.kernel,
    out_type=jax.ShapeDtypeStruct((NC, NPAD, DH), jnp.float32),
    mesh=_mesh,
    scratch_types=[
        pltpu.VMEM((CPW, CH), jnp.int32),
        pltpu.VMEM((CPW, CH), jnp.int32),
        pltpu.VMEM((CPW, CH), jnp.float32),
        pltpu.VMEM((CH, DH), jnp.float32),
        pltpu.VMEM((CH, DH), jnp.float32),
        pltpu.VMEM((CH, DH), jnp.float32),
        pltpu.VMEM_SHARED((NPAD, DH), jnp.float32),
        pltpu.SemaphoreType.DMA,
        pltpu.SemaphoreType.DMA,
        pltpu.SemaphoreType.DMA,
        pltpu.SemaphoreType.DMA,
        pltpu.SemaphoreType.DMA,
        pltpu.SemaphoreType.DMA,
    ],
    compiler_params=pltpu.CompilerParams(use_tc_tiling_on_sc=False),
    interpret=interpret,
  )(_mp_body)


def _mp_body(hp_hbm, src_hbm, dst_hbm, w_hbm, out_hbm,
             srca, dsta, wa, rows0, rows1, rows2, acc,
             gsem0, gsem1, gsem2, ssem0, ssem1, ssem2):
    c = lax.axis_index("c")
    s = lax.axis_index("s")
    zero = jnp.zeros((LANE,), jnp.float32)

    def zrow(i, carry):
        for f in range(DH // LANE):
            rows0[i, pl.ds(f * LANE, LANE)] = zero
        return carry

    lax.fori_loop(0, CH, zrow, 0)
    for r in range(RPT // CH):
        pltpu.sync_copy(rows0, acc.at[pl.ds(s * RPT + r * CH, CH)])
    pltpu.sync_copy(rows0.at[pl.ds(0, RPT % CH)],
                    acc.at[pl.ds(s * RPT + RPT - RPT % CH, RPT % CH)])
    pltpu.sync_copy(src_hbm.at[s], srca)
    pltpu.sync_copy(dst_hbm.at[s], dsta)
    pltpu.sync_copy(w_hbm.at[s], wa)

    hp = hp_hbm.at[c]
    bufs = ((rows0, gsem0, ssem0), (rows1, gsem1, ssem1),
            (rows2, gsem2, ssem2))
    take_idx = [jnp.full((LANE,), jj, jnp.int32) for jj in range(LANE)]

    for k in range(2):
        pltpu.async_copy(hp.at[srca.at[k]], bufs[k][0], bufs[k][1])
    plsc.subcore_barrier()

    def scale(rows, j):
        @plsc.parallel_loop(0, CH // LANE, unroll=4)
        def grp(g):
            w16 = wa[j, pl.ds(g * LANE, LANE)]
            for jj in range(LANE):
                wb = w16[take_idx[jj]]
                e = g * LANE + jj
                for f in range(DH // LANE):
                    rows[e, pl.ds(f * LANE, LANE)] = (
                        rows[e, pl.ds(f * LANE, LANE)] * wb)

    def trip(q, carry):
        for b in range(3):
            j = 3 * q + b
            rows, gsem, ssem = bufs[b]
            prows, pgsem, pssem = bufs[(b + 2) % 3]
            pltpu.make_async_copy(hp.at[srca.at[j]], rows, gsem).wait()

            @pl.when(jnp.logical_and(j >= 1, j + 2 < CPW))
            def _():
                pltpu.make_async_copy(
                    prows, acc.at[dsta.at[j - 1]], pssem).wait()

            @pl.when(j + 2 < CPW)
            def _():
                pltpu.async_copy(hp.at[srca.at[j + 2]], prows, pgsem)

            scale(rows, j)
            pltpu.async_copy(rows, acc.at[dsta.at[j]], ssem, add=True)
        return carry

    lax.fori_loop(0, CPW // 3, trip, 0)
    for k in range(3 * (CPW // 3), CPW):
        rows, gsem, ssem = bufs[k % 3]
        pltpu.make_async_copy(hp.at[srca.at[k]], rows, gsem).wait()
        scale(rows, k)
        pltpu.sync_copy(rows, acc.at[dsta.at[k]], add=True)
    for k in range(CPW - 3, 3 * (CPW // 3)):
        rows, _, ssem = bufs[k % 3]
        pltpu.make_async_copy(rows, acc.at[dsta.at[k]], ssem).wait()
    plsc.subcore_barrier()
    pltpu.sync_copy(acc.at[pl.ds(s * RPT, RPT)],
                    out_hbm.at[c, pl.ds(s * RPT, RPT)])


_deg_kernel = _make_deg()
_mp_kernel = _make_mp()


def _tc1_body(degp_ref, x_ref, w1_ref, dis_ref, hp_ref):
    deg = degp_ref[0, :, 0:1] + degp_ref[1, :, 0:1] + 1.0
    dis = jnp.where(deg > 0, lax.rsqrt(jnp.maximum(deg, 1e-12)), 0.0)
    h = jnp.dot(x_ref[...], w1_ref[...], preferred_element_type=jnp.float32)
    hp = h * dis
    dis_ref[...] = dis
    hp_ref[0] = hp[:, :DH]
    hp_ref[1] = hp[:, DH:]


_tc1 = pl.pallas_call(
    _tc1_body,
    grid=(GRID,),
    in_specs=[
        pl.BlockSpec((NC, BN, LANE), lambda i: (0, i, 0)),
        pl.BlockSpec((BN, D), lambda i: (i, 0)),
        pl.BlockSpec((D, D), lambda i: (0, 0)),
    ],
    out_specs=[
        pl.BlockSpec((BN, 1), lambda i: (i, 0)),
        pl.BlockSpec((NC, BN, DH), lambda i: (0, i, 0)),
    ],
    out_shape=[
        jax.ShapeDtypeStruct((N, 1), jnp.float32),
        jax.ShapeDtypeStruct((NC, N, DH), jnp.float32),
    ],
)


def _tc2_body(p_ref, hp_ref, dis_ref, b_ref, w2_ref, hp2_ref):
    ssum = jnp.concatenate(
        [p_ref[0] + hp_ref[0], p_ref[1] + hp_ref[1]], axis=1)
    o = jnp.maximum(dis_ref[...] * ssum + b_ref[...], 0.0)
    h2 = jnp.dot(o, w2_ref[...], preferred_element_type=jnp.float32)
    hp2 = h2 * dis_ref[...]
    hp2_ref[0] = hp2[:, :DH]
    hp2_ref[1] = hp2[:, DH:]


_tc2 = pl.pallas_call(
    _tc2_body,
    grid=(GRID,),
    in_specs=[
        pl.BlockSpec((NC, BN, DH), lambda i: (0, i, 0)),
        pl.BlockSpec((NC, BN, DH), lambda i: (0, i, 0)),
        pl.BlockSpec((BN, 1), lambda i: (i, 0)),
        pl.BlockSpec((1, D), lambda i: (0, 0)),
        pl.BlockSpec((D, D), lambda i: (0, 0)),
    ],
    out_specs=pl.BlockSpec((NC, BN, DH), lambda i: (0, i, 0)),
    out_shape=jax.ShapeDtypeStruct((NC, N, DH), jnp.float32),
)


def _tc3_body(p_ref, hp_ref, dis_ref, b_ref, out_ref):
    ssum = jnp.concatenate(
        [p_ref[0] + hp_ref[0], p_ref[1] + hp_ref[1]], axis=1)
    out_ref[...] = dis_ref[...] * ssum + b_ref[...]


_tc3 = pl.pallas_call(
    _tc3_body,
    grid=(GRID,),
    in_specs=[
        pl.BlockSpec((NC, BN, DH), lambda i: (0, i, 0)),
        pl.BlockSpec((NC, BN, DH), lambda i: (0, i, 0)),
        pl.BlockSpec((BN, 1), lambda i: (i, 0)),
        pl.BlockSpec((1, D), lambda i: (0, 0)),
    ],
    out_specs=pl.BlockSpec((BN, D), lambda i: (i, 0)),
    out_shape=jax.ShapeDtypeStruct((N, D), jnp.float32),
)


def kernel(x, edge_index, edge_attr, W1, b1, W2, b2):
    src = edge_index[0]
    dst = edge_index[1]
    pad = E_PAD - src.shape[0]
    src_p = jnp.concatenate([src, jnp.zeros((pad,), src.dtype)])
    dst_p = jnp.concatenate([dst, jnp.zeros((pad,), dst.dtype)])
    w_p = jnp.concatenate([edge_attr, jnp.zeros((pad,), edge_attr.dtype)])
    w_bc = jnp.broadcast_to(w_p[:, None], (E_PAD, LANE))
    src3 = src_p.reshape(NS, CPW, CH)
    dst3 = dst_p.reshape(NS, CPW, CH)
    w3 = w_p.reshape(NS, CPW, CH)
    degp = _deg_kernel(dst3, w_bc)
    dis, hp1 = _tc1(degp, x, W1)
    p1 = _mp_kernel(hp1, src3, dst3, w3)
    hp2 = _tc2(p1, hp1, dis, b1.reshape(1, D), W2)
    p2 = _mp_kernel(hp2, src3, dst3, w3)
    return _tc3(p2, hp2, dis, b2.reshape(1, D))
